# Initial kernel scaffold; baseline (speedup 1.0000x reference)
import jax
import jax.numpy as jnp
from jax.experimental import pallas as pl


def kernel(edge_embedding, nbr_idx):
    raise NotImplementedError("write your pallas kernel here")


# speedup vs baseline: 1.8207x; 1.8207x over previous
import functools

import jax
import jax.numpy as jnp
from jax import lax
from jax.experimental import pallas as pl
from jax.experimental.pallas import tpu as pltpu
from jax.experimental.pallas import tpu_sc as plsc

B, AT, NBR, F = 8, 96, 16, 128
K = NBR - 1
NT = B * AT * NBR
NOUT = NT * K
NW = 32
TRIP_W = NT // NW
ROWS_W = TRIP_W * K
GROUPS = TRIP_W // NBR
CHUNK = 128
NCHUNK = ROWS_W // CHUNK


@functools.partial(
    pl.kernel,
    mesh=plsc.VectorSubcoreMesh(core_axis_name="c", subcore_axis_name="s"),
    out_type=jax.ShapeDtypeStruct((NOUT, F), jnp.float32),
    compiler_params=pltpu.CompilerParams(needs_layout_passes=False),
    scratch_types=[
        pltpu.VMEM((TRIP_W,), jnp.int32),
        pltpu.VMEM((ROWS_W,), jnp.int32),
        pltpu.VMEM((CHUNK, F), jnp.float32),
        pltpu.VMEM((CHUNK, F), jnp.float32),
        pltpu.SemaphoreType.DMA,
        pltpu.SemaphoreType.DMA,
    ],
)
def _gather_kernel(table_hbm, nbr_hbm, out_hbm, nbr_v, idx_v, stage_a, stage_b,
                   sem_a, sem_b):
    cid = lax.axis_index("c")
    sid = lax.axis_index("s")
    wid = sid * 2 + cid
    trip_base = wid * TRIP_W
    row_base = wid * ROWS_W
    mol = trip_base // (AT * NBR)

    pltpu.sync_copy(nbr_hbm.at[pl.ds(trip_base, TRIP_W)], nbr_v)

    iota = lax.iota(jnp.int32, 16)

    def build_vec(v, carry):
        p = v * 16 + iota
        t = lax.shift_right_logical(p * 34953, 19)
        s = p - t * K
        j = lax.bitwise_and(t, NBR - 1)
        a = plsc.load_gather(nbr_v, [t])
        ge = 1 + lax.shift_right_arithmetic(s - j, 31)
        idx_v[pl.ds(v * 16, 16)] = (a + mol * AT) * NBR + s + ge
        return carry

    lax.fori_loop(0, ROWS_W // 16, build_vec, 0)

    def start(c, stage, sem):
        pltpu.async_copy(table_hbm.at[idx_v.at[pl.ds(c * CHUNK, CHUNK)]], stage, sem)

    def wait(c, stage, sem):
        pltpu.make_async_copy(
            table_hbm.at[idx_v.at[pl.ds(c * CHUNK, CHUNK)]], stage, sem
        ).wait()

    def copy_out(c, stage):
        pltpu.sync_copy(stage, out_hbm.at[pl.ds(row_base + c * CHUNK, CHUNK)])

    start(0, stage_a, sem_a)

    def chunk_step(h, carry):
        c = h * 2
        wait(c, stage_a, sem_a)
        start(c + 1, stage_b, sem_b)
        copy_out(c, stage_a)
        wait(c + 1, stage_b, sem_b)
        start(c + 2, stage_a, sem_a)
        copy_out(c + 1, stage_b)
        return carry

    lax.fori_loop(0, (NCHUNK - 1) // 2, chunk_step, 0)

    wait(NCHUNK - 1, stage_a, sem_a)
    copy_out(NCHUNK - 1, stage_a)


def kernel(edge_embedding, nbr_idx):
    table = edge_embedding.reshape(NT, F)
    nbr_flat = nbr_idx.reshape(NT)
    out = _gather_kernel(table, nbr_flat)
    return out.reshape(B, AT, NBR, K, F)


# speedup vs baseline: 2.9053x; 1.5957x over previous
import functools

import jax
import jax.numpy as jnp
from jax import lax
from jax.experimental import pallas as pl
from jax.experimental.pallas import tpu as pltpu
from jax.experimental.pallas import tpu_sc as plsc

B, AT, NBR, F = 8, 96, 16, 128
K = NBR - 1
NT = B * AT * NBR
NW = 32
TRIP_W = NT // NW
ROWS_W = TRIP_W * K
CH_T = 8
CH_R = CH_T * K
NCH = TRIP_W // CH_T


@functools.partial(
    pl.kernel,
    mesh=plsc.VectorSubcoreMesh(core_axis_name="c", subcore_axis_name="s"),
    out_type=jax.ShapeDtypeStruct((NT, K, F), jnp.float32),
    compiler_params=pltpu.CompilerParams(needs_layout_passes=False),
    scratch_types=[
        pltpu.VMEM((TRIP_W,), jnp.int32),
        pltpu.VMEM((ROWS_W,), jnp.int32),
        pltpu.VMEM((CH_R, F), jnp.float32),
        pltpu.VMEM((CH_R, F), jnp.float32),
        pltpu.SemaphoreType.DMA,
        pltpu.SemaphoreType.DMA,
        pltpu.SemaphoreType.DMA,
        pltpu.SemaphoreType.DMA,
    ],
)
def _gather_kernel(table_hbm, nbr_hbm, out_hbm, nbr_v, idx_v, stage_a, stage_b,
                   gsem_a, gsem_b, wsem_a, wsem_b):
    cid = lax.axis_index("c")
    sid = lax.axis_index("s")
    wid = sid * 2 + cid
    trip_base = wid * TRIP_W
    mol = trip_base // (AT * NBR)

    pltpu.sync_copy(nbr_hbm.at[pl.ds(trip_base, TRIP_W)], nbr_v)

    iota = lax.iota(jnp.int32, 16)

    def build_vec(v, carry):
        p = v * 16 + iota
        t = lax.shift_right_logical(p * 34953, 19)
        s = p - t * K
        j = lax.bitwise_and(t, NBR - 1)
        a = plsc.load_gather(nbr_v, [t])
        ge = 1 + lax.shift_right_arithmetic(s - j, 31)
        idx_v[pl.ds(v * 16, 16)] = (a + mol * AT) * NBR + s + ge
        return carry

    lax.fori_loop(0, ROWS_W // 16, build_vec, 0)

    def g_start(c, stage, sem):
        pltpu.async_copy(table_hbm.at[idx_v.at[pl.ds(c * CH_R, CH_R)]], stage, sem)

    def g_wait(c, stage, sem):
        pltpu.make_async_copy(
            table_hbm.at[idx_v.at[pl.ds(c * CH_R, CH_R)]], stage, sem
        ).wait()

    def w_fire(c, stage, sem):
        t0 = trip_base + c * CH_T
        for r in range(CH_T):
            pltpu.async_copy(stage.at[pl.ds(r * K, K)], out_hbm.at[t0 + r], sem)

    def w_drain(c, stage, sem):
        t0 = trip_base + c * CH_T
        for r in range(CH_T):
            pltpu.make_async_copy(
                stage.at[pl.ds(r * K, K)], out_hbm.at[t0 + r], sem
            ).wait()

    g_start(0, stage_a, gsem_a)
    g_start(1, stage_b, gsem_b)

    def pair_step(h, carry):
        c0 = h * 2
        g_wait(c0, stage_a, gsem_a)
        w_fire(c0, stage_a, wsem_a)
        g_wait(c0 + 1, stage_b, gsem_b)
        w_fire(c0 + 1, stage_b, wsem_b)
        w_drain(c0, stage_a, wsem_a)
        g_start(c0 + 2, stage_a, gsem_a)
        w_drain(c0 + 1, stage_b, wsem_b)
        g_start(c0 + 3, stage_b, gsem_b)
        return carry

    lax.fori_loop(0, NCH // 2 - 1, pair_step, 0)

    c_last = NCH - 2
    g_wait(c_last, stage_a, gsem_a)
    w_fire(c_last, stage_a, wsem_a)
    g_wait(c_last + 1, stage_b, gsem_b)
    w_fire(c_last + 1, stage_b, wsem_b)
    w_drain(c_last, stage_a, wsem_a)
    w_drain(c_last + 1, stage_b, wsem_b)


def kernel(edge_embedding, nbr_idx):
    table = edge_embedding.reshape(NT, F)
    nbr_flat = nbr_idx.reshape(NT)
    out = _gather_kernel(table, nbr_flat)
    return out.reshape(B, AT, NBR, K, F)
